# baseline (speedup 1.0000x reference)
import functools

import jax
import jax.numpy as jnp
from jax import lax
from jax.experimental import pallas as pl
from jax.experimental.pallas import tpu as pltpu
from jax.experimental.pallas import tpu_sc as plsc

NC = 2
NS = 16
L = 16
K = 40
NBUF = 2


def _edge_pass(xl, xr, src, dst, att):
    n, d = xl.shape
    e = src.shape[0]
    nw = NC * NS
    ew = e // nw
    nchunk = ew // K
    assert nchunk % NBUF == 0
    rpt = (n // NS) // 8 * 8
    tail = n - NS * rpt
    nj = d // L

    mesh = plsc.VectorSubcoreMesh(core_axis_name="c", subcore_axis_name="s",
                                  num_cores=NC, num_subcores=NS)

    scratch = [
        pltpu.VMEM_SHARED((n, d), jnp.float32),
        pltpu.VMEM((n + L,), jnp.float32),
        pltpu.VMEM((d,), jnp.float32),
    ]
    for _ in range(NBUF):
        scratch += [
            pltpu.VMEM((K,), jnp.int32),
            pltpu.VMEM((K,), jnp.int32),
            pltpu.VMEM((K,), jnp.int32),
            pltpu.VMEM((K, d), jnp.float32),
            pltpu.VMEM((K, d), jnp.float32),
            pltpu.VMEM((K, d), jnp.float32),
            pltpu.SemaphoreType.DMA,
            pltpu.SemaphoreType.DMA,
            pltpu.SemaphoreType.DMA,
            pltpu.SemaphoreType.DMA,
            pltpu.SemaphoreType.DMA,
            pltpu.SemaphoreType.DMA,
        ]

    @functools.partial(
        pl.kernel,
        out_type=(
            jax.ShapeDtypeStruct((NC, n, d), jnp.float32),
            jax.ShapeDtypeStruct((nw, 1, n), jnp.float32),
        ),
        mesh=mesh,
        scratch_types=scratch,
    )
    def body(xl_h, xr_h, src_h, dst_h, att_h, num_h, z_h,
             num_sh, ztile, attv, *bufs):
        src_v = [bufs[12 * b + 0] for b in range(NBUF)]
        dst_v = [bufs[12 * b + 1] for b in range(NBUF)]
        sdst_v = [bufs[12 * b + 2] for b in range(NBUF)]
        xlv = [bufs[12 * b + 3] for b in range(NBUF)]
        xrv = [bufs[12 * b + 4] for b in range(NBUF)]
        msgv = [bufs[12 * b + 5] for b in range(NBUF)]
        gsem = [bufs[12 * b + 6] for b in range(NBUF)]
        gsem2 = [bufs[12 * b + 7] for b in range(NBUF)]
        ssem = [bufs[12 * b + 8] for b in range(NBUF)]
        isem1 = [bufs[12 * b + 9] for b in range(NBUF)]
        isem2 = [bufs[12 * b + 10] for b in range(NBUF)]
        isem3 = [bufs[12 * b + 11] for b in range(NBUF)]

        cid = lax.axis_index("c")
        sid = lax.axis_index("s")
        wid = sid * NC + cid
        ebase = wid * ew
        last = ebase + (nchunk - 1) * K

        def zrow(i, carry):
            for j in range(nj):
                msgv[0][i, pl.ds(L * j, L)] = jnp.zeros((L,), jnp.float32)
            return carry
        lax.fori_loop(0, K, zrow, 0)

        def zz(i, carry):
            ztile[pl.ds(i * L, L)] = jnp.zeros((L,), jnp.float32)
            return carry
        lax.fori_loop(0, (n + L) // L, zz, 0)
        row0 = sid * rpt
        for t in range(rpt // K):
            pltpu.sync_copy(msgv[0], num_sh.at[pl.ds(row0 + K * t, K)])
        rem = rpt - (rpt // K) * K
        if rem:
            pltpu.sync_copy(msgv[0].at[pl.ds(0, rem)],
                            num_sh.at[pl.ds(row0 + (rpt // K) * K, rem)])

        @pl.when(sid == NS - 1)
        def _zero_tail():
            pltpu.sync_copy(msgv[0].at[pl.ds(0, tail)],
                            num_sh.at[pl.ds(NS * rpt, tail)])
        plsc.subcore_barrier()

        pltpu.sync_copy(att_h, attv)
        att_regs = [attv[pl.ds(L * j, L)] for j in range(nj)]
        lanes = lax.iota(jnp.int32, L)
        bfly = [jnp.bitwise_xor(lanes, sh) for sh in (8, 4, 2, 1)]
        fzero = jnp.zeros((L,), jnp.float32)

        def compute_chunk(b, zidx):
            def edge(k, didx, j):
                rows = []
                acc = jnp.zeros((L,), jnp.float32)
                for jj in range(nj):
                    a = xlv[b][k, pl.ds(L * jj, L)]
                    bb = xrv[b][k, pl.ds(L * jj, L)]
                    t = a + bb
                    h = jnp.maximum(t, 0.2 * t)
                    acc = acc + h * att_regs[jj]
                    rows.append(a)
                for idx in bfly:
                    acc = acc + acc.at[idx].get(mode="promise_in_bounds")
                p = jnp.exp(acc)
                for jj in range(nj):
                    msgv[b][k, pl.ds(L * jj, L)] = rows[jj] * p
                dk = didx[j]
                zbase = (dk // L) * L
                posv = jnp.full((L,), dk - zbase, jnp.int32)
                padd = jnp.where(lanes == posv, p, fzero)
                ztile[pl.ds(zbase, L)] = ztile[pl.ds(zbase, L)] + padd

            def group(gg, c2):
                didx = zidx[pl.ds(gg * L, L)]
                for j in range(L):
                    edge(gg * L + j, didx, j)
                return c2
            lax.fori_loop(0, K // L, group, 0)
            ktail = (K // L) * L
            if ktail < K:
                didx = zidx[pl.ds(K - L, L)]
                for j in range(K - ktail):
                    edge(ktail + j, didx, ktail + j - (K - L))

        pltpu.sync_copy(src_h.at[pl.ds(ebase, K)], src_v[0])
        pltpu.sync_copy(dst_h.at[pl.ds(ebase, K)], dst_v[0])
        pltpu.async_copy(xl_h.at[src_v[0]], xlv[0], gsem[0])
        pltpu.async_copy(xr_h.at[dst_v[0]], xrv[0], gsem2[0])

        def pair(g, carry):
            for b in range(NBUF):
                bp = 1 - b
                c = NBUF * g + b

                @pl.when(g >= 1)
                def _drain():
                    pltpu.make_async_copy(
                        msgv[b], num_sh.at[sdst_v[b]], ssem[b]).wait()
                pltpu.make_async_copy(xl_h.at[src_v[b]], xlv[b],
                                      gsem[b]).wait()
                pltpu.make_async_copy(xr_h.at[dst_v[b]], xrv[b],
                                      gsem2[b]).wait()
                nbase = jnp.minimum(ebase + (c + 1) * K, last)
                di1 = pltpu.async_copy(src_h.at[pl.ds(nbase, K)],
                                       src_v[bp], isem1[bp])
                di2 = pltpu.async_copy(dst_h.at[pl.ds(nbase, K)],
                                       dst_v[bp], isem2[bp])
                di3 = pltpu.async_copy(dst_h.at[pl.ds(ebase + c * K, K)],
                                       sdst_v[b], isem3[b])
                compute_chunk(b, dst_v[b])
                di3.wait()
                pltpu.async_copy(msgv[b], num_sh.at[sdst_v[b]], ssem[b],
                                 add=True)
                di1.wait()
                di2.wait()
                pltpu.async_copy(xl_h.at[src_v[bp]], xlv[bp], gsem[bp])
                pltpu.async_copy(xr_h.at[dst_v[bp]], xrv[bp], gsem2[bp])
            return carry
        lax.fori_loop(0, nchunk // NBUF, pair, 0)

        pltpu.make_async_copy(xl_h.at[src_v[0]], xlv[0], gsem[0]).wait()
        pltpu.make_async_copy(xr_h.at[dst_v[0]], xrv[0], gsem2[0]).wait()
        for b in range(NBUF):
            pltpu.make_async_copy(msgv[b], num_sh.at[sdst_v[b]],
                                  ssem[b]).wait()
        plsc.subcore_barrier()

        pltpu.sync_copy(num_sh.at[pl.ds(row0, rpt)],
                        num_h.at[cid, pl.ds(row0, rpt)])
        pltpu.sync_copy(ztile.at[pl.ds(0, n)], z_h.at[wid, 0])

        @pl.when(sid == NS - 1)
        def _read_tail():
            pltpu.sync_copy(num_sh.at[pl.ds(NS * rpt, tail)],
                            num_h.at[cid, pl.ds(NS * rpt, tail)])

    return body(xl, xr, src, dst, att)


_ROWS = 1000


def _proj2(x, Wl, bl, Wr, br):
    n, d = x.shape

    def body(x_ref, wl_ref, bl_ref, wr_ref, br_ref, xl_ref, xr_ref):
        xx = x_ref[...]
        xl_ref[...] = jnp.dot(xx, wl_ref[...],
                              precision=lax.Precision.HIGHEST) + bl_ref[...]
        xr_ref[...] = jnp.dot(xx, wr_ref[...],
                              precision=lax.Precision.HIGHEST) + br_ref[...]

    return pl.pallas_call(
        body,
        grid=(n // _ROWS,),
        in_specs=[
            pl.BlockSpec((_ROWS, d), lambda i: (i, 0)),
            pl.BlockSpec((d, d), lambda i: (0, 0)),
            pl.BlockSpec((1, d), lambda i: (0, 0)),
            pl.BlockSpec((d, d), lambda i: (0, 0)),
            pl.BlockSpec((1, d), lambda i: (0, 0)),
        ],
        out_specs=[
            pl.BlockSpec((_ROWS, d), lambda i: (i, 0)),
            pl.BlockSpec((_ROWS, d), lambda i: (i, 0)),
        ],
        out_shape=[jax.ShapeDtypeStruct((n, d), jnp.float32)] * 2,
    )(x, Wl, bl, Wr, br)


def _combine_proj2(num, z, bias, Wl, bl, Wr, br):
    _, n, d = num.shape
    nw = z.shape[1]

    def body(np_ref, zp_ref, bias_ref, wl_ref, bl_ref, wr_ref, br_ref,
             xl_ref, xr_ref):
        acc = np_ref[0] + np_ref[1]
        zz = jnp.sum(zp_ref[...], axis=1, keepdims=True)
        h = acc / (zz + 1e-30) + bias_ref[...]
        h = jnp.maximum(h, 0.0)
        xl_ref[...] = jnp.dot(h, wl_ref[...],
                              precision=lax.Precision.HIGHEST) + bl_ref[...]
        xr_ref[...] = jnp.dot(h, wr_ref[...],
                              precision=lax.Precision.HIGHEST) + br_ref[...]

    return pl.pallas_call(
        body,
        grid=(n // _ROWS,),
        in_specs=[
            pl.BlockSpec((NC, _ROWS, d), lambda i: (0, i, 0)),
            pl.BlockSpec((_ROWS, nw), lambda i: (i, 0)),
            pl.BlockSpec((1, d), lambda i: (0, 0)),
            pl.BlockSpec((d, d), lambda i: (0, 0)),
            pl.BlockSpec((1, d), lambda i: (0, 0)),
            pl.BlockSpec((d, d), lambda i: (0, 0)),
            pl.BlockSpec((1, d), lambda i: (0, 0)),
        ],
        out_specs=[
            pl.BlockSpec((_ROWS, d), lambda i: (i, 0)),
            pl.BlockSpec((_ROWS, d), lambda i: (i, 0)),
        ],
        out_shape=[jax.ShapeDtypeStruct((n, d), jnp.float32)] * 2,
    )(num, z, bias, Wl, bl, Wr, br)


def _combine_out(num, z, bias, W, b):
    _, n, d = num.shape
    nw = z.shape[1]
    dout = W.shape[1]

    def body(np_ref, zp_ref, bias_ref, w_ref, b_ref, o_ref):
        acc = np_ref[0] + np_ref[1]
        zz = jnp.sum(zp_ref[...], axis=1, keepdims=True)
        h = acc / (zz + 1e-30) + bias_ref[...]
        o_ref[...] = jnp.dot(h, w_ref[...],
                             precision=lax.Precision.HIGHEST) + b_ref[...]

    return pl.pallas_call(
        body,
        grid=(n // _ROWS,),
        in_specs=[
            pl.BlockSpec((NC, _ROWS, d), lambda i: (0, i, 0)),
            pl.BlockSpec((_ROWS, nw), lambda i: (i, 0)),
            pl.BlockSpec((1, d), lambda i: (0, 0)),
            pl.BlockSpec((d, dout), lambda i: (0, 0)),
            pl.BlockSpec((1, dout), lambda i: (0, 0)),
        ],
        out_specs=pl.BlockSpec((_ROWS, dout), lambda i: (i, 0)),
        out_shape=jax.ShapeDtypeStruct((n, dout), jnp.float32),
    )(num, z, bias, W, b)


def kernel(x, edge_index, W1l, b1l, W1r, b1r, att1, bias1,
           W2l, b2l, W2r, b2r, att2, bias2, Wout, bout):
    src = edge_index[0]
    dst = edge_index[1]
    xl1, xr1 = _proj2(x, W1l, b1l.reshape(1, -1), W1r, b1r.reshape(1, -1))
    num1, z1 = _edge_pass(xl1, xr1, src, dst, att1.reshape(-1))
    z1t = z1.reshape(z1.shape[0], -1).T
    xl2, xr2 = _combine_proj2(num1, z1t, bias1.reshape(1, -1),
                              W2l, b2l.reshape(1, -1),
                              W2r, b2r.reshape(1, -1))
    num2, z2 = _edge_pass(xl2, xr2, src, dst, att2.reshape(-1))
    z2t = z2.reshape(z2.shape[0], -1).T
    return _combine_out(num2, z2t, bias2.reshape(1, -1),
                        Wout, bout.reshape(1, -1))


# speedup vs baseline: 1.5211x; 1.5211x over previous
import functools

import jax
import jax.numpy as jnp
from jax import lax
from jax.experimental import pallas as pl
from jax.experimental.pallas import tpu as pltpu
from jax.experimental.pallas import tpu_sc as plsc

NC = 2
NS = 16
L = 16
K = 80
NBUF = 2


def _edge_pass(xl, xr, src, dst, att):
    n, d = xl.shape
    e = src.shape[0]
    nw = NC * NS
    ew = e // nw
    nchunk = ew // K
    rpt = (n // NS) // 8 * 8
    tail = n - NS * rpt
    nj = d // L

    mesh = plsc.VectorSubcoreMesh(core_axis_name="c", subcore_axis_name="s",
                                  num_cores=NC, num_subcores=NS)

    scratch = [
        pltpu.VMEM_SHARED((n, d), jnp.float32),
        pltpu.VMEM((n,), jnp.float32),
        pltpu.VMEM((d,), jnp.float32),
        pltpu.VMEM((K, d), jnp.float32),
        pltpu.SemaphoreType.DMA,
    ]
    for _ in range(NBUF):
        scratch += [
            pltpu.VMEM((K,), jnp.int32),
            pltpu.VMEM((K,), jnp.int32),
            pltpu.VMEM((K,), jnp.int32),
            pltpu.VMEM((K, d), jnp.float32),
            pltpu.SemaphoreType.DMA,
            pltpu.SemaphoreType.DMA,
            pltpu.SemaphoreType.DMA,
            pltpu.SemaphoreType.DMA,
            pltpu.SemaphoreType.DMA,
        ]

    @functools.partial(
        pl.kernel,
        out_type=(
            jax.ShapeDtypeStruct((NC, n, d), jnp.float32),
            jax.ShapeDtypeStruct((nw, 1, n), jnp.float32),
        ),
        mesh=mesh,
        scratch_types=scratch,
    )
    def body(xl_h, xr_h, src_h, dst_h, att_h, num_h, z_h,
             num_sh, ztile, attv, xrv, xsem, *bufs):
        src_v = [bufs[9 * b + 0] for b in range(NBUF)]
        dst_v = [bufs[9 * b + 1] for b in range(NBUF)]
        sdst_v = [bufs[9 * b + 2] for b in range(NBUF)]
        xlv = [bufs[9 * b + 3] for b in range(NBUF)]
        gsem = [bufs[9 * b + 4] for b in range(NBUF)]
        ssem = [bufs[9 * b + 5] for b in range(NBUF)]
        isem1 = [bufs[9 * b + 6] for b in range(NBUF)]
        isem2 = [bufs[9 * b + 7] for b in range(NBUF)]
        isem3 = [bufs[9 * b + 8] for b in range(NBUF)]

        cid = lax.axis_index("c")
        sid = lax.axis_index("s")
        wid = sid * NC + cid
        ebase = wid * ew
        last = ebase + (nchunk - 1) * K

        def zrow(i, carry):
            for j in range(nj):
                xlv[0][i, pl.ds(L * j, L)] = jnp.zeros((L,), jnp.float32)
            return carry
        lax.fori_loop(0, K, zrow, 0)

        def zz(i, carry):
            ztile[pl.ds(i * L, L)] = jnp.zeros((L,), jnp.float32)
            return carry
        lax.fori_loop(0, n // L, zz, 0)
        row0 = sid * rpt
        for t in range(rpt // K):
            pltpu.sync_copy(xlv[0], num_sh.at[pl.ds(row0 + K * t, K)])
        rem = rpt - (rpt // K) * K
        if rem:
            pltpu.sync_copy(xlv[0].at[pl.ds(0, rem)],
                            num_sh.at[pl.ds(row0 + (rpt // K) * K, rem)])

        @pl.when(sid == NS - 1)
        def _zero_tail():
            pltpu.sync_copy(xlv[0].at[pl.ds(0, tail)],
                            num_sh.at[pl.ds(NS * rpt, tail)])
        plsc.subcore_barrier()

        pltpu.sync_copy(att_h, attv)
        att_regs = [attv[pl.ds(L * j, L)] for j in range(nj)]
        lanes = lax.iota(jnp.int32, L)
        bfly = [jnp.bitwise_xor(lanes, sh) for sh in (8, 4, 2, 1)]
        fzero = jnp.zeros((L,), jnp.float32)

        def compute_chunk(b, zidx):
            def edge(k, didx, j):
                rows = []
                acc = jnp.zeros((L,), jnp.float32)
                for jj in range(nj):
                    a = xlv[b][k, pl.ds(L * jj, L)]
                    bb = xrv[k, pl.ds(L * jj, L)]
                    t = a + bb
                    h = jnp.maximum(t, 0.2 * t)
                    acc = acc + h * att_regs[jj]
                    rows.append(a)
                for idx in bfly:
                    acc = acc + acc.at[idx].get(mode="promise_in_bounds")
                p = jnp.exp(acc)
                for jj in range(nj):
                    xlv[b][k, pl.ds(L * jj, L)] = rows[jj] * p
                dk = didx[j]
                zbase = (dk // L) * L
                posv = jnp.full((L,), dk - zbase, jnp.int32)
                padd = jnp.where(lanes == posv, p, fzero)
                ztile[pl.ds(zbase, L)] = ztile[pl.ds(zbase, L)] + padd

            def group(gg, c2):
                didx = zidx[pl.ds(gg * L, L)]
                for j in range(L):
                    edge(gg * L + j, didx, j)
                return c2
            lax.fori_loop(0, K // L, group, 0)
            ktail = (K // L) * L
            if ktail < K:
                didx = zidx[pl.ds(K - L, L)]
                for j in range(K - ktail):
                    edge(ktail + j, didx, ktail + j - (K - L))

        pltpu.sync_copy(src_h.at[pl.ds(ebase, K)], src_v[0])
        pltpu.sync_copy(dst_h.at[pl.ds(ebase, K)], dst_v[0])

        def section(c, g, b, last_sec):
            bp = 1 - b

            @pl.when(g >= 1)
            def _drain():
                pltpu.make_async_copy(
                    xlv[b], num_sh.at[sdst_v[b]], ssem[b]).wait()
            pltpu.async_copy(xl_h.at[src_v[b]], xlv[b], gsem[b])

            def wait_prev():
                pltpu.make_async_copy(xl_h.at[src_v[bp]], xlv[bp],
                                      gsem[bp]).wait()
                pltpu.make_async_copy(xr_h.at[dst_v[bp]], xrv, xsem).wait()

            def run_prev():
                compute_chunk(bp, sdst_v[bp])
                pltpu.async_copy(xlv[bp], num_sh.at[sdst_v[bp]], ssem[bp],
                                 add=True)
            if b == 0:
                @pl.when(g >= 1)
                def _w():
                    wait_prev()
            else:
                wait_prev()
            if not last_sec:
                di1 = pltpu.async_copy(
                    src_h.at[pl.ds(ebase + (c + 1) * K, K)],
                    src_v[bp], isem1[bp])
                di2 = pltpu.async_copy(
                    dst_h.at[pl.ds(ebase + (c + 1) * K, K)],
                    dst_v[bp], isem2[bp])
            di3 = pltpu.async_copy(dst_h.at[pl.ds(ebase + c * K, K)],
                                   sdst_v[b], isem3[b])
            if b == 0:
                @pl.when(g >= 1)
                def _p():
                    run_prev()
            else:
                run_prev()
            pltpu.async_copy(xr_h.at[dst_v[b]], xrv, xsem)
            if not last_sec:
                di1.wait()
                di2.wait()
            di3.wait()

        def pair(g, carry):
            for b in range(NBUF):
                section(NBUF * g + b, g, b, False)
            return carry
        lax.fori_loop(0, (nchunk - 1) // NBUF, pair, 0)
        section(nchunk - 1, (nchunk - 1) // NBUF, 0, True)

        pltpu.make_async_copy(xl_h.at[src_v[0]], xlv[0], gsem[0]).wait()
        pltpu.make_async_copy(xr_h.at[dst_v[0]], xrv, xsem).wait()
        pltpu.make_async_copy(xlv[1], num_sh.at[sdst_v[1]], ssem[1]).wait()
        compute_chunk(0, sdst_v[0])
        pltpu.sync_copy(xlv[0], num_sh.at[sdst_v[0]], add=True)
        plsc.subcore_barrier()

        pltpu.sync_copy(num_sh.at[pl.ds(row0, rpt)],
                        num_h.at[cid, pl.ds(row0, rpt)])
        pltpu.sync_copy(ztile.at[pl.ds(0, n)], z_h.at[wid, 0])

        @pl.when(sid == NS - 1)
        def _read_tail():
            pltpu.sync_copy(num_sh.at[pl.ds(NS * rpt, tail)],
                            num_h.at[cid, pl.ds(NS * rpt, tail)])

    return body(xl, xr, src, dst, att)


_ROWS = 1000


def _proj2(x, Wl, bl, Wr, br):
    n, d = x.shape

    def body(x_ref, wl_ref, bl_ref, wr_ref, br_ref, xl_ref, xr_ref):
        xx = x_ref[...]
        xl_ref[...] = jnp.dot(xx, wl_ref[...],
                              precision=lax.Precision.HIGHEST) + bl_ref[...]
        xr_ref[...] = jnp.dot(xx, wr_ref[...],
                              precision=lax.Precision.HIGHEST) + br_ref[...]

    return pl.pallas_call(
        body,
        grid=(n // _ROWS,),
        in_specs=[
            pl.BlockSpec((_ROWS, d), lambda i: (i, 0)),
            pl.BlockSpec((d, d), lambda i: (0, 0)),
            pl.BlockSpec((1, d), lambda i: (0, 0)),
            pl.BlockSpec((d, d), lambda i: (0, 0)),
            pl.BlockSpec((1, d), lambda i: (0, 0)),
        ],
        out_specs=[
            pl.BlockSpec((_ROWS, d), lambda i: (i, 0)),
            pl.BlockSpec((_ROWS, d), lambda i: (i, 0)),
        ],
        out_shape=[jax.ShapeDtypeStruct((n, d), jnp.float32)] * 2,
    )(x, Wl, bl, Wr, br)


def _combine_proj2(num, z, bias, Wl, bl, Wr, br):
    _, n, d = num.shape
    nw = z.shape[1]

    def body(np_ref, zp_ref, bias_ref, wl_ref, bl_ref, wr_ref, br_ref,
             xl_ref, xr_ref):
        acc = np_ref[0] + np_ref[1]
        zz = jnp.sum(zp_ref[...], axis=1, keepdims=True)
        h = acc / (zz + 1e-30) + bias_ref[...]
        h = jnp.maximum(h, 0.0)
        xl_ref[...] = jnp.dot(h, wl_ref[...],
                              precision=lax.Precision.HIGHEST) + bl_ref[...]
        xr_ref[...] = jnp.dot(h, wr_ref[...],
                              precision=lax.Precision.HIGHEST) + br_ref[...]

    return pl.pallas_call(
        body,
        grid=(n // _ROWS,),
        in_specs=[
            pl.BlockSpec((NC, _ROWS, d), lambda i: (0, i, 0)),
            pl.BlockSpec((_ROWS, nw), lambda i: (i, 0)),
            pl.BlockSpec((1, d), lambda i: (0, 0)),
            pl.BlockSpec((d, d), lambda i: (0, 0)),
            pl.BlockSpec((1, d), lambda i: (0, 0)),
            pl.BlockSpec((d, d), lambda i: (0, 0)),
            pl.BlockSpec((1, d), lambda i: (0, 0)),
        ],
        out_specs=[
            pl.BlockSpec((_ROWS, d), lambda i: (i, 0)),
            pl.BlockSpec((_ROWS, d), lambda i: (i, 0)),
        ],
        out_shape=[jax.ShapeDtypeStruct((n, d), jnp.float32)] * 2,
    )(num, z, bias, Wl, bl, Wr, br)


def _combine_out(num, z, bias, W, b):
    _, n, d = num.shape
    nw = z.shape[1]
    dout = W.shape[1]

    def body(np_ref, zp_ref, bias_ref, w_ref, b_ref, o_ref):
        acc = np_ref[0] + np_ref[1]
        zz = jnp.sum(zp_ref[...], axis=1, keepdims=True)
        h = acc / (zz + 1e-30) + bias_ref[...]
        o_ref[...] = jnp.dot(h, w_ref[...],
                             precision=lax.Precision.HIGHEST) + b_ref[...]

    return pl.pallas_call(
        body,
        grid=(n // _ROWS,),
        in_specs=[
            pl.BlockSpec((NC, _ROWS, d), lambda i: (0, i, 0)),
            pl.BlockSpec((_ROWS, nw), lambda i: (i, 0)),
            pl.BlockSpec((1, d), lambda i: (0, 0)),
            pl.BlockSpec((d, dout), lambda i: (0, 0)),
            pl.BlockSpec((1, dout), lambda i: (0, 0)),
        ],
        out_specs=pl.BlockSpec((_ROWS, dout), lambda i: (i, 0)),
        out_shape=jax.ShapeDtypeStruct((n, dout), jnp.float32),
    )(num, z, bias, W, b)


def kernel(x, edge_index, W1l, b1l, W1r, b1r, att1, bias1,
           W2l, b2l, W2r, b2r, att2, bias2, Wout, bout):
    src = edge_index[0]
    dst = edge_index[1]
    xl1, xr1 = _proj2(x, W1l, b1l.reshape(1, -1), W1r, b1r.reshape(1, -1))
    num1, z1 = _edge_pass(xl1, xr1, src, dst, att1.reshape(-1))
    z1t = z1.reshape(z1.shape[0], -1).T
    xl2, xr2 = _combine_proj2(num1, z1t, bias1.reshape(1, -1),
                              W2l, b2l.reshape(1, -1),
                              W2r, b2r.reshape(1, -1))
    num2, z2 = _edge_pass(xl2, xr2, src, dst, att2.reshape(-1))
    z2t = z2.reshape(z2.shape[0], -1).T
    return _combine_out(num2, z2t, bias2.reshape(1, -1),
                        Wout, bout.reshape(1, -1))
